# Initial kernel scaffold; baseline (speedup 1.0000x reference)
import jax
import jax.numpy as jnp
from jax.experimental import pallas as pl


def kernel(feat, feat_a, adj, graph_neigh, W1, a_src1, a_dst1, W2, a_src2, a_dst2, bn1_gamma, bn1_beta, bn2_gamma, bn2_beta, Wb, bb):
    raise NotImplementedError("write your pallas kernel here")


# speedup vs baseline: 1.0563x; 1.0563x over previous
import jax
import jax.numpy as jnp
from jax.experimental import pallas as pl
from jax.experimental.pallas import tpu as pltpu

_N = 10000
_BM = 512
_BK = 2048
_BN_EPS = 1e-5


def _read_body(mask_ref, hi_ref, lo_ref, out_ref, acc_ref):
    k = pl.program_id(1)
    nk = pl.num_programs(1)

    @pl.when(k == 0)
    def _():
        acc_ref[...] = jnp.zeros_like(acc_ref)

    mb = mask_ref[...]
    col = jax.lax.broadcasted_iota(jnp.int32, mb.shape, 1) + k * _BK
    mb = jnp.where(col < _N, mb, 0.0)
    mb16 = mb.astype(jnp.bfloat16)
    acc_ref[...] += (
        jnp.dot(mb16, hi_ref[...], preferred_element_type=jnp.float32)
        + jnp.dot(mb16, lo_ref[...], preferred_element_type=jnp.float32)
    )

    @pl.when(k == nk - 1)
    def _():
        out_ref[...] = acc_ref[...]


def _readout(mask, embcat):
    npad = 5 * _BK
    hi = embcat.astype(jnp.bfloat16)
    lo = (embcat - hi.astype(jnp.float32)).astype(jnp.bfloat16)
    hi_ext = jnp.zeros((npad, 256), jnp.bfloat16)
    hi_ext = hi_ext.at[:_N, :128].set(hi).at[:_N, 128].set(1.0)
    lo_ext = jnp.zeros((npad, 256), jnp.bfloat16).at[:_N, :128].set(lo)
    grid = (pl.cdiv(_N, _BM), npad // _BK)
    out = pl.pallas_call(
        _read_body,
        grid=grid,
        in_specs=[
            pl.BlockSpec((_BM, _BK), lambda i, k: (i, k)),
            pl.BlockSpec((_BK, 256), lambda i, k: (k, 0)),
            pl.BlockSpec((_BK, 256), lambda i, k: (k, 0)),
        ],
        out_specs=pl.BlockSpec((_BM, 256), lambda i, k: (i, 0)),
        out_shape=jax.ShapeDtypeStruct((_N, 256), jnp.float32),
        scratch_shapes=[pltpu.VMEM((_BM, 256), jnp.float32)],
    )(mask, hi_ext, lo_ext)
    return out[:, :128], out[:, 128]


def _gat_edges(h, s_src, d_dst, src, dst):
    e = s_src[src] + d_dst[dst]
    e = jnp.where(e >= 0, e, 0.2 * e)
    ex = jnp.exp(e)
    den = jax.ops.segment_sum(ex, dst, num_segments=_N)
    num = jax.ops.segment_sum(ex[:, None] * h[src], dst, num_segments=_N)
    return num, den


def _bn(x, gamma, beta):
    return x / jnp.sqrt(1.0 + _BN_EPS) * gamma + beta


def kernel(feat, feat_a, adj, graph_neigh, W1, a_src1, a_dst1, W2, a_src2,
           a_dst2, bn1_gamma, bn1_beta, bn2_gamma, bn2_beta, Wb, bb):
    src, dst = adj[0], adj[1]

    h1 = feat @ W1
    h3 = feat_a @ W1
    num1, den1 = _gat_edges(h1, h1 @ a_src1, h1 @ a_dst1, src, dst)
    num3, den3 = _gat_edges(h3, h3 @ a_src1, h3 @ a_dst1, src, dst)
    z = _bn(num1 / (den1 + 1e-16)[:, None], bn1_gamma, bn1_beta)
    z_a = _bn(num3 / (den3 + 1e-16)[:, None], bn1_gamma, bn1_beta)
    hiden_emb = z

    h2 = z @ W2
    num2, den2 = _gat_edges(h2, h2 @ a_src2, h2 @ a_dst2, src, dst)
    h_out = _bn(num2 / (den2 + 1e-16)[:, None], bn2_gamma, bn2_beta)

    emb = jax.nn.relu(z)
    emb_a = jax.nn.relu(z_a)
    embcat = jnp.concatenate([emb, emb_a], axis=1)

    vsum, rs = _readout(graph_neigh, embcat)
    gb = vsum / rs[:, None]
    n1 = jnp.maximum(jnp.linalg.norm(gb[:, :64], axis=1, keepdims=True), 1e-12)
    n2 = jnp.maximum(jnp.linalg.norm(gb[:, 64:], axis=1, keepdims=True), 1e-12)
    g = jax.nn.sigmoid(gb[:, :64] / n1)
    g_a = jax.nn.sigmoid(gb[:, 64:] / n2)

    t1 = emb @ Wb
    t2 = emb_a @ Wb
    ret = jnp.stack([jnp.sum(t1 * g, 1), jnp.sum(t2 * g, 1)], axis=1) + bb[0]
    ret_a = jnp.stack([jnp.sum(t2 * g_a, 1), jnp.sum(t1 * g_a, 1)], axis=1) + bb[0]
    return (hiden_emb, h_out, ret, ret_a)


# speedup vs baseline: 1.3216x; 1.2511x over previous
import dataclasses
import functools

import jax
import jax.numpy as jnp
from jax import lax
from jax.experimental import pallas as pl
from jax.experimental.pallas import tpu as pltpu
from jax.experimental.pallas import tpu_sc as plsc

_N = 10000
_E = 160000
_BM = 512
_BK = 2048
_BN_EPS = 1e-5

_NW = 32
_TILE = 64
_SV = _TILE // 16
_T = 80
_EPW = _T * _TILE
_EPAD = _NW * _EPW
_CT = 8


def _make_edge_kernel(two: bool):
    ne = 2 if two else 1
    mesh = plsc.VectorSubcoreMesh(core_axis_name="c", subcore_axis_name="s")
    cp = pltpu.CompilerParams()
    if "needs_layout_passes" in pltpu.CompilerParams.__dataclass_fields__:
        cp = dataclasses.replace(cp, needs_layout_passes=False)

    @functools.partial(
        pl.kernel,
        out_type=(jax.ShapeDtypeStruct((2 * _N, 128), jnp.float32),
                  jax.ShapeDtypeStruct((_NW * ne * _N,), jnp.float32)),
        mesh=mesh,
        compiler_params=cp,
        scratch_types=[
            pltpu.VMEM((_CT, _TILE), jnp.int32),
            pltpu.VMEM((_CT, _TILE), jnp.int32),
            pltpu.VMEM((_TILE, 128), jnp.float32),
            pltpu.VMEM((_TILE,), jnp.float32),
            pltpu.VMEM((_TILE,), jnp.float32),
            pltpu.SemaphoreType.DMA,
            pltpu.VMEM_SHARED((_N, 128), jnp.float32),
        ] + [pltpu.VMEM((_CT * _TILE,), jnp.float32)] * ne
          + [pltpu.VMEM((_N,), jnp.float32)] * ne,
    )
    def edge_kernel(h_hbm, e_hbm, src_hbm, dst_hbm, out_hbm, den_hbm,
                    srcb, dstb, growsb, exb1, exb3, gsem, acc, *rest):
        eb = rest[:ne]
        denb = rest[ne:]
        cid = lax.axis_index("c")
        sid = lax.axis_index("s")
        wid = cid * 16 + sid

        zv = jnp.zeros((16,), jnp.float32)

        @pl.loop(0, _TILE)
        def _(r):
            for cg in range(8):
                growsb[r, pl.ds(cg * 16, 16)] = zv

        @pl.loop(0, _N // 16)
        def _(i):
            for g in range(ne):
                denb[g][pl.ds(i * 16, 16)] = zv

        @pl.loop(0, 9)
        def _(i):
            pltpu.sync_copy(growsb, acc.at[pl.ds(sid * 624 + i * 64, 64)])
        pltpu.sync_copy(growsb.at[pl.ds(0, 48)],
                        acc.at[pl.ds(sid * 624 + 576, 48)])

        @pl.when(sid == 15)
        def _():
            pltpu.sync_copy(growsb.at[pl.ds(0, 16)], acc.at[pl.ds(9984, 16)])

        plsc.subcore_barrier()

        ebase = wid * _EPW
        iota = lax.iota(jnp.int32, 16)
        cpe = _CT * _TILE

        @pl.loop(0, _T // _CT)
        def _(c):
            pltpu.sync_copy(src_hbm.at[wid * (_T // _CT) + c], srcb)
            pltpu.sync_copy(dst_hbm.at[wid * (_T // _CT) + c], dstb)
            for g in range(ne):
                pltpu.sync_copy(
                    e_hbm.at[pl.ds(g * _EPAD + ebase + c * cpe, cpe)], eb[g])

            @pl.loop(0, _CT)
            def _(tt):
                pltpu.async_copy(h_hbm.at[srcb.at[tt]], growsb, gsem).wait()

                for v in range(_SV):
                    sl = pl.ds(v * 16, 16)
                    esl = pl.ds(tt * _TILE + v * 16, 16)
                    valid = (ebase + c * cpe + tt * _TILE + v * 16 + iota) < _E
                    dstv = dstb[tt, sl]
                    e1 = eb[0][esl]
                    e1 = jnp.where(e1 >= 0.0, e1, 0.2 * e1)
                    ex1 = jnp.where(valid, jnp.exp(e1), 0.0)
                    exb1[sl] = ex1
                    plsc.addupdate_scatter(denb[0], [dstv], ex1)
                    if two:
                        e3 = eb[1][esl]
                        e3 = jnp.where(e3 >= 0.0, e3, 0.2 * e3)
                        ex3 = jnp.where(valid, jnp.exp(e3), 0.0)
                        exb3[sl] = ex3
                        plsc.addupdate_scatter(denb[1], [dstv], ex3)

                @pl.loop(0, _TILE)
                def _(r):
                    rsp = jnp.full((16,), r, jnp.int32)
                    sp1 = plsc.load_gather(exb1, [rsp])
                    if two:
                        sp3 = plsc.load_gather(exb3, [rsp])
                        for cg in range(4):
                            growsb[r, pl.ds(cg * 16, 16)] = (
                                growsb[r, pl.ds(cg * 16, 16)] * sp1)
                        for cg in range(4, 8):
                            growsb[r, pl.ds(cg * 16, 16)] = (
                                growsb[r, pl.ds(cg * 16, 16)] * sp3)
                    else:
                        for cg in range(8):
                            growsb[r, pl.ds(cg * 16, 16)] = (
                                growsb[r, pl.ds(cg * 16, 16)] * sp1)

                pltpu.sync_copy(growsb, acc.at[dstb.at[tt]], add=True)

        for g in range(ne):
            pltpu.sync_copy(denb[g],
                            den_hbm.at[pl.ds((wid * ne + g) * _N, _N)])

        plsc.subcore_barrier()

        @pl.loop(0, 3)
        def _(i):
            off = sid * 624 + i * 208
            pltpu.sync_copy(acc.at[pl.ds(off, 208)],
                            out_hbm.at[pl.ds(cid * _N + off, 208)])

        @pl.when(sid == 15)
        def _():
            pltpu.sync_copy(acc.at[pl.ds(9984, 16)],
                            out_hbm.at[pl.ds(cid * _N + 9984, 16)])

    return edge_kernel


_EDGE2 = _make_edge_kernel(True)
_EDGE1 = _make_edge_kernel(False)


def _read_body(mask_ref, hi_ref, lo_ref, out_ref, acc_ref):
    k = pl.program_id(1)
    nk = pl.num_programs(1)

    @pl.when(k == 0)
    def _():
        acc_ref[...] = jnp.zeros_like(acc_ref)

    mb = mask_ref[...]
    col = jax.lax.broadcasted_iota(jnp.int32, mb.shape, 1) + k * _BK
    mb = jnp.where(col < _N, mb, 0.0)
    mb16 = mb.astype(jnp.bfloat16)
    acc_ref[...] += (
        jnp.dot(mb16, hi_ref[...], preferred_element_type=jnp.float32)
        + jnp.dot(mb16, lo_ref[...], preferred_element_type=jnp.float32)
    )

    @pl.when(k == nk - 1)
    def _():
        out_ref[...] = acc_ref[...]


def _readout(mask, embcat):
    npad = 5 * _BK
    hi = embcat.astype(jnp.bfloat16)
    lo = (embcat - hi.astype(jnp.float32)).astype(jnp.bfloat16)
    hi_ext = jnp.zeros((npad, 256), jnp.bfloat16)
    hi_ext = hi_ext.at[:_N, :128].set(hi).at[:_N, 128].set(1.0)
    lo_ext = jnp.zeros((npad, 256), jnp.bfloat16).at[:_N, :128].set(lo)
    grid = (pl.cdiv(_N, _BM), npad // _BK)
    out = pl.pallas_call(
        _read_body,
        grid=grid,
        in_specs=[
            pl.BlockSpec((_BM, _BK), lambda i, k: (i, k)),
            pl.BlockSpec((_BK, 256), lambda i, k: (k, 0)),
            pl.BlockSpec((_BK, 256), lambda i, k: (k, 0)),
        ],
        out_specs=pl.BlockSpec((_BM, 256), lambda i, k: (i, 0)),
        out_shape=jax.ShapeDtypeStruct((_N, 256), jnp.float32),
        scratch_shapes=[pltpu.VMEM((_BM, 256), jnp.float32)],
    )(mask, hi_ext, lo_ext)
    return out[:, :128], out[:, 128]


def _bn(x, gamma, beta):
    return x / jnp.sqrt(1.0 + _BN_EPS) * gamma + beta


def _pade(v):
    return jnp.pad(v, (0, _EPAD - _E))


def kernel(feat, feat_a, adj, graph_neigh, W1, a_src1, a_dst1, W2, a_src2,
           a_dst2, bn1_gamma, bn1_beta, bn2_gamma, bn2_beta, Wb, bb):
    src, dst = adj[0], adj[1]
    pad = jnp.zeros((_EPAD - _E,), jnp.int32)
    nch = _NW * (_T // _CT)
    srcg = jnp.concatenate([src, pad]).reshape(nch, _CT, _TILE)
    dstg = jnp.concatenate([dst, pad]).reshape(nch, _CT, _TILE)

    h1 = feat @ W1
    h3 = feat_a @ W1
    hcat = jnp.concatenate([h1, h3], axis=1)
    e1 = (h1 @ a_src1)[src] + (h1 @ a_dst1)[dst]
    e3 = (h3 @ a_src1)[src] + (h3 @ a_dst1)[dst]
    e13 = jnp.concatenate([_pade(e1), _pade(e3)])
    out13, dens13 = _EDGE2(hcat, e13, srcg, dstg)
    agg = out13[:_N] + out13[_N:]
    d13 = dens13.reshape(_NW, 2, _N).sum(axis=0)
    den1 = d13[0] + 1e-16
    den3 = d13[1] + 1e-16
    z = _bn(agg[:, :64] / den1[:, None], bn1_gamma, bn1_beta)
    z_a = _bn(agg[:, 64:128] / den3[:, None], bn1_gamma, bn1_beta)
    hiden_emb = z

    h2 = z @ W2
    e2 = (h2 @ a_src2)[src] + (h2 @ a_dst2)[dst]
    out2, dens2 = _EDGE1(h2, _pade(e2), srcg, dstg)
    agg2 = out2[:_N] + out2[_N:]
    den2 = dens2.reshape(_NW, _N).sum(axis=0) + 1e-16
    h_out = _bn(agg2 / den2[:, None], bn2_gamma, bn2_beta)

    emb = jax.nn.relu(z)
    emb_a = jax.nn.relu(z_a)
    embcat = jnp.concatenate([emb, emb_a], axis=1)

    vsum, rs = _readout(graph_neigh, embcat)
    gb = vsum / rs[:, None]
    n1 = jnp.maximum(jnp.linalg.norm(gb[:, :64], axis=1, keepdims=True), 1e-12)
    n2 = jnp.maximum(jnp.linalg.norm(gb[:, 64:], axis=1, keepdims=True), 1e-12)
    g = jax.nn.sigmoid(gb[:, :64] / n1)
    g_a = jax.nn.sigmoid(gb[:, 64:] / n2)

    t1 = emb @ Wb
    t2 = emb_a @ Wb
    ret = jnp.stack([jnp.sum(t1 * g, 1), jnp.sum(t2 * g, 1)], axis=1) + bb[0]
    ret_a = jnp.stack([jnp.sum(t2 * g_a, 1), jnp.sum(t1 * g_a, 1)], axis=1) + bb[0]
    return (hiden_emb, h_out, ret, ret_a)


# speedup vs baseline: 10.6165x; 8.0332x over previous
import dataclasses
import functools

import jax
import jax.numpy as jnp
from jax import lax
from jax.experimental import pallas as pl
from jax.experimental.pallas import tpu as pltpu
from jax.experimental.pallas import tpu_sc as plsc

_N = 10000
_E = 160000
_BM = 512
_BK = 2048
_BN_EPS = 1e-5

_NW = 32
_TILE = 64
_SV = _TILE // 16
_T = 80
_EPW = _T * _TILE
_EPAD = _NW * _EPW
_CT = 8


def _make_edge_kernel(two: bool):
    ne = 2 if two else 1
    mesh = plsc.VectorSubcoreMesh(core_axis_name="c", subcore_axis_name="s")
    cp = pltpu.CompilerParams()
    if "needs_layout_passes" in pltpu.CompilerParams.__dataclass_fields__:
        cp = dataclasses.replace(cp, needs_layout_passes=False)

    @functools.partial(
        pl.kernel,
        out_type=(jax.ShapeDtypeStruct((2 * _N, 128), jnp.float32),
                  jax.ShapeDtypeStruct((_NW * ne * _N,), jnp.float32)),
        mesh=mesh,
        compiler_params=cp,
        scratch_types=[
            pltpu.VMEM((_CT, _TILE), jnp.int32),
            pltpu.VMEM((_CT, _TILE), jnp.int32),
            pltpu.VMEM((_TILE, 128), jnp.float32),
            pltpu.VMEM((_TILE,), jnp.float32),
            pltpu.VMEM((_TILE,), jnp.float32),
            pltpu.SemaphoreType.DMA,
            pltpu.SemaphoreType.DMA,
            pltpu.VMEM_SHARED((_N, 128), jnp.float32),
        ] + [pltpu.VMEM((_TILE,), jnp.float32)] * (2 * ne)
          + [pltpu.VMEM((_N,), jnp.float32)] * ne,
    )
    def edge_kernel(h_hbm, *args):
        sd_hbm = args[:2 * ne]
        src_hbm, dst_hbm, out_hbm, den_hbm = args[2 * ne:2 * ne + 4]
        (srcb, dstb, growsb, exb1, exb3, gsem, ssem,
         acc) = args[2 * ne + 4:2 * ne + 12]
        sdb = args[2 * ne + 12:2 * ne + 12 + 2 * ne]
        denb = args[2 * ne + 12 + 2 * ne:]
        cid = lax.axis_index("c")
        sid = lax.axis_index("s")
        wid = cid * 16 + sid

        zv = jnp.zeros((16,), jnp.float32)

        @pl.loop(0, _TILE)
        def _(r):
            for cg in range(8):
                growsb[r, pl.ds(cg * 16, 16)] = zv

        @pl.loop(0, _N // 16)
        def _(i):
            for g in range(ne):
                denb[g][pl.ds(i * 16, 16)] = zv

        @pl.loop(0, 9)
        def _(i):
            pltpu.sync_copy(growsb, acc.at[pl.ds(sid * 624 + i * 64, 64)])
        pltpu.sync_copy(growsb.at[pl.ds(0, 48)],
                        acc.at[pl.ds(sid * 624 + 576, 48)])

        @pl.when(sid == 15)
        def _():
            pltpu.sync_copy(growsb.at[pl.ds(0, 16)], acc.at[pl.ds(9984, 16)])

        plsc.subcore_barrier()

        ebase = wid * _EPW
        iota = lax.iota(jnp.int32, 16)
        cpe = _CT * _TILE

        @pl.loop(0, _T // _CT)
        def _(c):
            pltpu.sync_copy(src_hbm.at[wid * (_T // _CT) + c], srcb)
            pltpu.sync_copy(dst_hbm.at[wid * (_T // _CT) + c], dstb)

            @pl.loop(0, _CT)
            def _(tt):
                pltpu.async_copy(h_hbm.at[srcb.at[tt]], growsb, gsem)
                for g in range(ne):
                    pltpu.async_copy(sd_hbm[2 * g].at[srcb.at[tt]],
                                     sdb[2 * g], ssem)
                    pltpu.async_copy(sd_hbm[2 * g + 1].at[dstb.at[tt]],
                                     sdb[2 * g + 1], ssem)
                pltpu.make_async_copy(h_hbm.at[srcb.at[tt]], growsb,
                                      gsem).wait()
                for g in range(ne):
                    pltpu.make_async_copy(sd_hbm[2 * g].at[srcb.at[tt]],
                                          sdb[2 * g], ssem).wait()
                    pltpu.make_async_copy(sd_hbm[2 * g + 1].at[dstb.at[tt]],
                                          sdb[2 * g + 1], ssem).wait()

                for v in range(_SV):
                    sl = pl.ds(v * 16, 16)
                    valid = (ebase + c * cpe + tt * _TILE + v * 16 + iota) < _E
                    dstv = dstb[tt, sl]
                    e1 = sdb[0][sl] + sdb[1][sl]
                    e1 = jnp.where(e1 >= 0.0, e1, 0.2 * e1)
                    ex1 = jnp.where(valid, jnp.exp(e1), 0.0)
                    exb1[sl] = ex1
                    plsc.addupdate_scatter(denb[0], [dstv], ex1)
                    if two:
                        e3 = sdb[2][sl] + sdb[3][sl]
                        e3 = jnp.where(e3 >= 0.0, e3, 0.2 * e3)
                        ex3 = jnp.where(valid, jnp.exp(e3), 0.0)
                        exb3[sl] = ex3
                        plsc.addupdate_scatter(denb[1], [dstv], ex3)

                @pl.loop(0, _TILE)
                def _(r):
                    rsp = jnp.full((16,), r, jnp.int32)
                    sp1 = plsc.load_gather(exb1, [rsp])
                    if two:
                        sp3 = plsc.load_gather(exb3, [rsp])
                        for cg in range(4):
                            growsb[r, pl.ds(cg * 16, 16)] = (
                                growsb[r, pl.ds(cg * 16, 16)] * sp1)
                        for cg in range(4, 8):
                            growsb[r, pl.ds(cg * 16, 16)] = (
                                growsb[r, pl.ds(cg * 16, 16)] * sp3)
                    else:
                        for cg in range(8):
                            growsb[r, pl.ds(cg * 16, 16)] = (
                                growsb[r, pl.ds(cg * 16, 16)] * sp1)

                pltpu.sync_copy(growsb, acc.at[dstb.at[tt]], add=True)

        for g in range(ne):
            pltpu.sync_copy(denb[g],
                            den_hbm.at[pl.ds((wid * ne + g) * _N, _N)])

        plsc.subcore_barrier()

        @pl.loop(0, 3)
        def _(i):
            off = sid * 624 + i * 208
            pltpu.sync_copy(acc.at[pl.ds(off, 208)],
                            out_hbm.at[pl.ds(cid * _N + off, 208)])

        @pl.when(sid == 15)
        def _():
            pltpu.sync_copy(acc.at[pl.ds(9984, 16)],
                            out_hbm.at[pl.ds(cid * _N + 9984, 16)])

    return edge_kernel


_EDGE2 = _make_edge_kernel(True)
_EDGE1 = _make_edge_kernel(False)


def _read_body(mask_ref, hi_ref, lo_ref, out_ref, acc_ref):
    k = pl.program_id(1)
    nk = pl.num_programs(1)

    @pl.when(k == 0)
    def _():
        acc_ref[...] = jnp.zeros_like(acc_ref)

    mb = mask_ref[...]
    col = jax.lax.broadcasted_iota(jnp.int32, mb.shape, 1) + k * _BK
    mb = jnp.where(col < _N, mb, 0.0)
    mb16 = mb.astype(jnp.bfloat16)
    acc_ref[...] += (
        jnp.dot(mb16, hi_ref[...], preferred_element_type=jnp.float32)
        + jnp.dot(mb16, lo_ref[...], preferred_element_type=jnp.float32)
    )

    @pl.when(k == nk - 1)
    def _():
        out_ref[...] = acc_ref[...]


def _readout(mask, embcat):
    npad = 5 * _BK
    hi = embcat.astype(jnp.bfloat16)
    lo = (embcat - hi.astype(jnp.float32)).astype(jnp.bfloat16)
    hi_ext = jnp.zeros((npad, 256), jnp.bfloat16)
    hi_ext = hi_ext.at[:_N, :128].set(hi).at[:_N, 128].set(1.0)
    lo_ext = jnp.zeros((npad, 256), jnp.bfloat16).at[:_N, :128].set(lo)
    grid = (pl.cdiv(_N, _BM), npad // _BK)
    out = pl.pallas_call(
        _read_body,
        grid=grid,
        in_specs=[
            pl.BlockSpec((_BM, _BK), lambda i, k: (i, k)),
            pl.BlockSpec((_BK, 256), lambda i, k: (k, 0)),
            pl.BlockSpec((_BK, 256), lambda i, k: (k, 0)),
        ],
        out_specs=pl.BlockSpec((_BM, 256), lambda i, k: (i, 0)),
        out_shape=jax.ShapeDtypeStruct((_N, 256), jnp.float32),
        scratch_shapes=[pltpu.VMEM((_BM, 256), jnp.float32)],
    )(mask, hi_ext, lo_ext)
    return out[:, :128], out[:, 128]


def _bn(x, gamma, beta):
    return x / jnp.sqrt(1.0 + _BN_EPS) * gamma + beta


def kernel(feat, feat_a, adj, graph_neigh, W1, a_src1, a_dst1, W2, a_src2,
           a_dst2, bn1_gamma, bn1_beta, bn2_gamma, bn2_beta, Wb, bb):
    src, dst = adj[0], adj[1]
    pad = jnp.zeros((_EPAD - _E,), jnp.int32)
    nch = _NW * (_T // _CT)
    srcg = jnp.concatenate([src, pad]).reshape(nch, _CT, _TILE)
    dstg = jnp.concatenate([dst, pad]).reshape(nch, _CT, _TILE)

    h1 = feat @ W1
    h3 = feat_a @ W1
    hcat = jnp.concatenate([h1, h3], axis=1)
    out13, dens13 = _EDGE2(hcat, h1 @ a_src1, h1 @ a_dst1,
                           h3 @ a_src1, h3 @ a_dst1, srcg, dstg)
    agg = out13[:_N] + out13[_N:]
    d13 = dens13.reshape(_NW, 2, _N).sum(axis=0)
    den1 = d13[0] + 1e-16
    den3 = d13[1] + 1e-16
    z = _bn(agg[:, :64] / den1[:, None], bn1_gamma, bn1_beta)
    z_a = _bn(agg[:, 64:128] / den3[:, None], bn1_gamma, bn1_beta)
    hiden_emb = z

    h2 = z @ W2
    out2, dens2 = _EDGE1(h2, h2 @ a_src2, h2 @ a_dst2, srcg, dstg)
    agg2 = out2[:_N] + out2[_N:]
    den2 = dens2.reshape(_NW, _N).sum(axis=0) + 1e-16
    h_out = _bn(agg2 / den2[:, None], bn2_gamma, bn2_beta)

    emb = jax.nn.relu(z)
    emb_a = jax.nn.relu(z_a)
    embcat = jnp.concatenate([emb, emb_a], axis=1)

    vsum, rs = _readout(graph_neigh, embcat)
    gb = vsum / rs[:, None]
    n1 = jnp.maximum(jnp.linalg.norm(gb[:, :64], axis=1, keepdims=True), 1e-12)
    n2 = jnp.maximum(jnp.linalg.norm(gb[:, 64:], axis=1, keepdims=True), 1e-12)
    g = jax.nn.sigmoid(gb[:, :64] / n1)
    g_a = jax.nn.sigmoid(gb[:, 64:] / n2)

    t1 = emb @ Wb
    t2 = emb_a @ Wb
    ret = jnp.stack([jnp.sum(t1 * g, 1), jnp.sum(t2 * g, 1)], axis=1) + bb[0]
    ret_a = jnp.stack([jnp.sum(t2 * g_a, 1), jnp.sum(t1 * g_a, 1)], axis=1) + bb[0]
    return (hiden_emb, h_out, ret, ret_a)


# speedup vs baseline: 11.7322x; 1.1051x over previous
import dataclasses
import functools

import jax
import jax.numpy as jnp
from jax import lax
from jax.experimental import pallas as pl
from jax.experimental.pallas import tpu as pltpu
from jax.experimental.pallas import tpu_sc as plsc

_N = 10000
_E = 160000
_BM = 512
_BK = 2048
_BN_EPS = 1e-5

_NW = 32
_TILE = 64
_SV = _TILE // 16
_T = 80
_EPW = _T * _TILE
_EPAD = _NW * _EPW
_CT = 8


def _make_edge_kernel(two: bool):
    ne = 2 if two else 1
    mesh = plsc.VectorSubcoreMesh(core_axis_name="c", subcore_axis_name="s")
    cp = pltpu.CompilerParams()
    if "needs_layout_passes" in pltpu.CompilerParams.__dataclass_fields__:
        cp = dataclasses.replace(cp, needs_layout_passes=False)

    @functools.partial(
        pl.kernel,
        out_type=(jax.ShapeDtypeStruct((2 * _N, 128), jnp.float32),
                  jax.ShapeDtypeStruct((_NW * ne * _N,), jnp.float32)),
        mesh=mesh,
        compiler_params=cp,
        scratch_types=[
            pltpu.VMEM((_CT, _TILE), jnp.int32),
            pltpu.VMEM((_CT, _TILE), jnp.int32),
            pltpu.VMEM((_TILE, 128), jnp.float32),
            pltpu.VMEM((_TILE, 128), jnp.float32),
            pltpu.VMEM((_TILE,), jnp.float32),
            pltpu.VMEM((_TILE,), jnp.float32),
            pltpu.SemaphoreType.DMA,
            pltpu.SemaphoreType.DMA,
            pltpu.VMEM_SHARED((_N, 128), jnp.float32),
        ] + [pltpu.VMEM((_TILE,), jnp.float32)] * (4 * ne)
          + [pltpu.VMEM((_N,), jnp.float32)] * ne,
    )
    def edge_kernel(h_hbm, *args):
        sd_hbm = args[:2 * ne]
        src_hbm, dst_hbm, out_hbm, den_hbm = args[2 * ne:2 * ne + 4]
        (srcb, dstb, grows0, grows1, exb1, exb3, gsem0, gsem1,
         acc) = args[2 * ne + 4:2 * ne + 13]
        sdb = args[2 * ne + 13:2 * ne + 13 + 4 * ne]
        denb = args[2 * ne + 13 + 4 * ne:]
        growsb = (grows0, grows1)
        gsem = (gsem0, gsem1)
        cid = lax.axis_index("c")
        sid = lax.axis_index("s")
        wid = cid * 16 + sid

        zv = jnp.zeros((16,), jnp.float32)

        @pl.loop(0, _TILE)
        def _(r):
            for cg in range(8):
                grows0[r, pl.ds(cg * 16, 16)] = zv

        @pl.loop(0, _N // 16)
        def _(i):
            for g in range(ne):
                denb[g][pl.ds(i * 16, 16)] = zv

        @pl.loop(0, 9)
        def _(i):
            pltpu.sync_copy(grows0, acc.at[pl.ds(sid * 624 + i * 64, 64)])
        pltpu.sync_copy(grows0.at[pl.ds(0, 48)],
                        acc.at[pl.ds(sid * 624 + 576, 48)])

        @pl.when(sid == 15)
        def _():
            pltpu.sync_copy(grows0.at[pl.ds(0, 16)], acc.at[pl.ds(9984, 16)])

        plsc.subcore_barrier()

        ebase = wid * _EPW
        iota = lax.iota(jnp.int32, 16)
        cpe = _CT * _TILE

        @pl.loop(0, _T // _CT)
        def _(c):
            pltpu.sync_copy(src_hbm.at[wid * (_T // _CT) + c], srcb)
            pltpu.sync_copy(dst_hbm.at[wid * (_T // _CT) + c], dstb)

            def fire(tt, s):
                pltpu.async_copy(h_hbm.at[srcb.at[tt]], growsb[s], gsem[s])
                for g in range(ne):
                    pltpu.async_copy(sd_hbm[2 * g].at[srcb.at[tt]],
                                     sdb[4 * g + 2 * s], gsem[s])
                    pltpu.async_copy(sd_hbm[2 * g + 1].at[dstb.at[tt]],
                                     sdb[4 * g + 2 * s + 1], gsem[s])

            def drain(tt, s):
                pltpu.make_async_copy(h_hbm.at[srcb.at[tt]], growsb[s],
                                      gsem[s]).wait()
                for g in range(ne):
                    pltpu.make_async_copy(sd_hbm[2 * g].at[srcb.at[tt]],
                                          sdb[4 * g + 2 * s], gsem[s]).wait()
                    pltpu.make_async_copy(sd_hbm[2 * g + 1].at[dstb.at[tt]],
                                          sdb[4 * g + 2 * s + 1],
                                          gsem[s]).wait()

            fire(0, 0)

            @pl.loop(0, _CT // 2)
            def _(p):
                for s in (0, 1):
                    tt = p * 2 + s
                    drain(tt, s)
                    @pl.when(tt + 1 < _CT)
                    def _():
                        fire(tt + 1, 1 - s)

                    for v in range(_SV):
                        sl = pl.ds(v * 16, 16)
                        valid = (ebase + c * cpe + tt * _TILE + v * 16
                                 + iota) < _E
                        dstv = dstb[tt, sl]
                        e1 = sdb[2 * s][sl] + sdb[2 * s + 1][sl]
                        e1 = jnp.where(e1 >= 0.0, e1, 0.2 * e1)
                        ex1 = jnp.where(valid, jnp.exp(e1), 0.0)
                        exb1[sl] = ex1
                        plsc.addupdate_scatter(denb[0], [dstv], ex1)
                        if two:
                            e3 = sdb[4 + 2 * s][sl] + sdb[4 + 2 * s + 1][sl]
                            e3 = jnp.where(e3 >= 0.0, e3, 0.2 * e3)
                            ex3 = jnp.where(valid, jnp.exp(e3), 0.0)
                            exb3[sl] = ex3
                            plsc.addupdate_scatter(denb[1], [dstv], ex3)

                    @pl.loop(0, _TILE)
                    def _(r, s=s):
                        rsp = jnp.full((16,), r, jnp.int32)
                        sp1 = plsc.load_gather(exb1, [rsp])
                        if two:
                            sp3 = plsc.load_gather(exb3, [rsp])
                            for cg in range(4):
                                growsb[s][r, pl.ds(cg * 16, 16)] = (
                                    growsb[s][r, pl.ds(cg * 16, 16)] * sp1)
                            for cg in range(4, 8):
                                growsb[s][r, pl.ds(cg * 16, 16)] = (
                                    growsb[s][r, pl.ds(cg * 16, 16)] * sp3)
                        else:
                            for cg in range(8):
                                growsb[s][r, pl.ds(cg * 16, 16)] = (
                                    growsb[s][r, pl.ds(cg * 16, 16)] * sp1)

                    pltpu.sync_copy(growsb[s], acc.at[dstb.at[tt]], add=True)

        for g in range(ne):
            pltpu.sync_copy(denb[g],
                            den_hbm.at[pl.ds((wid * ne + g) * _N, _N)])

        plsc.subcore_barrier()

        @pl.loop(0, 3)
        def _(i):
            off = sid * 624 + i * 208
            pltpu.sync_copy(acc.at[pl.ds(off, 208)],
                            out_hbm.at[pl.ds(cid * _N + off, 208)])

        @pl.when(sid == 15)
        def _():
            pltpu.sync_copy(acc.at[pl.ds(9984, 16)],
                            out_hbm.at[pl.ds(cid * _N + 9984, 16)])

    return edge_kernel


_EDGE2 = _make_edge_kernel(True)
_EDGE1 = _make_edge_kernel(False)


def _read_body(mask_ref, hi_ref, lo_ref, out_ref, acc_ref):
    k = pl.program_id(1)
    nk = pl.num_programs(1)

    @pl.when(k == 0)
    def _():
        acc_ref[...] = jnp.zeros_like(acc_ref)

    mb = mask_ref[...]
    col = jax.lax.broadcasted_iota(jnp.int32, mb.shape, 1) + k * _BK
    mb = jnp.where(col < _N, mb, 0.0)
    mb16 = mb.astype(jnp.bfloat16)
    acc_ref[...] += (
        jnp.dot(mb16, hi_ref[...], preferred_element_type=jnp.float32)
        + jnp.dot(mb16, lo_ref[...], preferred_element_type=jnp.float32)
    )

    @pl.when(k == nk - 1)
    def _():
        out_ref[...] = acc_ref[...]


def _readout(mask, embcat):
    npad = 5 * _BK
    hi = embcat.astype(jnp.bfloat16)
    lo = (embcat - hi.astype(jnp.float32)).astype(jnp.bfloat16)
    hi_ext = jnp.zeros((npad, 256), jnp.bfloat16)
    hi_ext = hi_ext.at[:_N, :128].set(hi).at[:_N, 128].set(1.0)
    lo_ext = jnp.zeros((npad, 256), jnp.bfloat16).at[:_N, :128].set(lo)
    grid = (pl.cdiv(_N, _BM), npad // _BK)
    out = pl.pallas_call(
        _read_body,
        grid=grid,
        in_specs=[
            pl.BlockSpec((_BM, _BK), lambda i, k: (i, k)),
            pl.BlockSpec((_BK, 256), lambda i, k: (k, 0)),
            pl.BlockSpec((_BK, 256), lambda i, k: (k, 0)),
        ],
        out_specs=pl.BlockSpec((_BM, 256), lambda i, k: (i, 0)),
        out_shape=jax.ShapeDtypeStruct((_N, 256), jnp.float32),
        scratch_shapes=[pltpu.VMEM((_BM, 256), jnp.float32)],
    )(mask, hi_ext, lo_ext)
    return out[:, :128], out[:, 128]


def _bn(x, gamma, beta):
    return x / jnp.sqrt(1.0 + _BN_EPS) * gamma + beta


def kernel(feat, feat_a, adj, graph_neigh, W1, a_src1, a_dst1, W2, a_src2,
           a_dst2, bn1_gamma, bn1_beta, bn2_gamma, bn2_beta, Wb, bb):
    src, dst = adj[0], adj[1]
    pad = jnp.zeros((_EPAD - _E,), jnp.int32)
    nch = _NW * (_T // _CT)
    srcg = jnp.concatenate([src, pad]).reshape(nch, _CT, _TILE)
    dstg = jnp.concatenate([dst, pad]).reshape(nch, _CT, _TILE)

    h1 = feat @ W1
    h3 = feat_a @ W1
    hcat = jnp.concatenate([h1, h3], axis=1)
    out13, dens13 = _EDGE2(hcat, h1 @ a_src1, h1 @ a_dst1,
                           h3 @ a_src1, h3 @ a_dst1, srcg, dstg)
    agg = out13[:_N] + out13[_N:]
    d13 = dens13.reshape(_NW, 2, _N).sum(axis=0)
    den1 = d13[0] + 1e-16
    den3 = d13[1] + 1e-16
    z = _bn(agg[:, :64] / den1[:, None], bn1_gamma, bn1_beta)
    z_a = _bn(agg[:, 64:128] / den3[:, None], bn1_gamma, bn1_beta)
    hiden_emb = z

    h2 = z @ W2
    out2, dens2 = _EDGE1(h2, h2 @ a_src2, h2 @ a_dst2, srcg, dstg)
    agg2 = out2[:_N] + out2[_N:]
    den2 = dens2.reshape(_NW, _N).sum(axis=0) + 1e-16
    h_out = _bn(agg2 / den2[:, None], bn2_gamma, bn2_beta)

    emb = jax.nn.relu(z)
    emb_a = jax.nn.relu(z_a)
    embcat = jnp.concatenate([emb, emb_a], axis=1)

    vsum, rs = _readout(graph_neigh, embcat)
    gb = vsum / rs[:, None]
    n1 = jnp.maximum(jnp.linalg.norm(gb[:, :64], axis=1, keepdims=True), 1e-12)
    n2 = jnp.maximum(jnp.linalg.norm(gb[:, 64:], axis=1, keepdims=True), 1e-12)
    g = jax.nn.sigmoid(gb[:, :64] / n1)
    g_a = jax.nn.sigmoid(gb[:, 64:] / n2)

    t1 = emb @ Wb
    t2 = emb_a @ Wb
    ret = jnp.stack([jnp.sum(t1 * g, 1), jnp.sum(t2 * g, 1)], axis=1) + bb[0]
    ret_a = jnp.stack([jnp.sum(t2 * g_a, 1), jnp.sum(t1 * g_a, 1)], axis=1) + bb[0]
    return (hiden_emb, h_out, ret, ret_a)


# speedup vs baseline: 11.7488x; 1.0014x over previous
import dataclasses
import functools

import jax
import jax.numpy as jnp
from jax import lax
from jax.experimental import pallas as pl
from jax.experimental.pallas import tpu as pltpu
from jax.experimental.pallas import tpu_sc as plsc

_N = 10000
_E = 160000
_BM = 512
_BK = 2048
_BN_EPS = 1e-5

_NW = 32
_TILE = 64
_SV = _TILE // 16
_T = 80
_EPW = _T * _TILE
_EPAD = _NW * _EPW
_CT = 8


def _make_edge_kernel(two: bool):
    ne = 2 if two else 1
    mesh = plsc.VectorSubcoreMesh(core_axis_name="c", subcore_axis_name="s")
    cp = pltpu.CompilerParams()
    if "needs_layout_passes" in pltpu.CompilerParams.__dataclass_fields__:
        cp = dataclasses.replace(cp, needs_layout_passes=False)

    @functools.partial(
        pl.kernel,
        out_type=(jax.ShapeDtypeStruct((2 * _N, 128), jnp.float32),
                  jax.ShapeDtypeStruct((_NW * ne * _N,), jnp.float32)),
        mesh=mesh,
        compiler_params=cp,
        scratch_types=[
            pltpu.VMEM((_CT, _TILE), jnp.int32),
            pltpu.VMEM((_CT, _TILE), jnp.int32),
            pltpu.VMEM((_TILE, 128), jnp.float32),
            pltpu.VMEM((_TILE, 128), jnp.float32),
            pltpu.VMEM((_TILE,), jnp.float32),
            pltpu.VMEM((_TILE,), jnp.float32),
            pltpu.SemaphoreType.DMA,
            pltpu.SemaphoreType.DMA,
            pltpu.SemaphoreType.DMA,
            pltpu.SemaphoreType.DMA,
            pltpu.VMEM_SHARED((_N, 128), jnp.float32),
        ] + [pltpu.VMEM((_TILE,), jnp.float32)] * (4 * ne)
          + [pltpu.VMEM((_N,), jnp.float32)] * ne,
    )
    def edge_kernel(h_hbm, *args):
        sd_hbm = args[:2 * ne]
        src_hbm, dst_hbm, out_hbm, den_hbm = args[2 * ne:2 * ne + 4]
        (srcb, dstb, grows0, grows1, exb1, exb3, gsem0, gsem1,
         ssem0, ssem1, acc) = args[2 * ne + 4:2 * ne + 15]
        sdb = args[2 * ne + 15:2 * ne + 15 + 4 * ne]
        denb = args[2 * ne + 15 + 4 * ne:]
        growsb = (grows0, grows1)
        gsem = (gsem0, gsem1)
        ssem = (ssem0, ssem1)
        cid = lax.axis_index("c")
        sid = lax.axis_index("s")
        wid = cid * 16 + sid

        zv = jnp.zeros((16,), jnp.float32)

        @pl.loop(0, _TILE)
        def _(r):
            for cg in range(8):
                grows0[r, pl.ds(cg * 16, 16)] = zv

        @pl.loop(0, _N // 16)
        def _(i):
            for g in range(ne):
                denb[g][pl.ds(i * 16, 16)] = zv

        @pl.loop(0, 9)
        def _(i):
            pltpu.sync_copy(grows0, acc.at[pl.ds(sid * 624 + i * 64, 64)])
        pltpu.sync_copy(grows0.at[pl.ds(0, 48)],
                        acc.at[pl.ds(sid * 624 + 576, 48)])

        @pl.when(sid == 15)
        def _():
            pltpu.sync_copy(grows0.at[pl.ds(0, 16)], acc.at[pl.ds(9984, 16)])

        plsc.subcore_barrier()

        ebase = wid * _EPW
        iota = lax.iota(jnp.int32, 16)
        cpe = _CT * _TILE

        @pl.loop(0, _T // _CT)
        def _(c):
            pltpu.sync_copy(src_hbm.at[wid * (_T // _CT) + c], srcb)
            pltpu.sync_copy(dst_hbm.at[wid * (_T // _CT) + c], dstb)

            def fire(tt, s):
                pltpu.async_copy(h_hbm.at[srcb.at[tt]], growsb[s], gsem[s])
                for g in range(ne):
                    pltpu.async_copy(sd_hbm[2 * g].at[srcb.at[tt]],
                                     sdb[4 * g + 2 * s], gsem[s])
                    pltpu.async_copy(sd_hbm[2 * g + 1].at[dstb.at[tt]],
                                     sdb[4 * g + 2 * s + 1], gsem[s])

            def drain(tt, s):
                pltpu.make_async_copy(h_hbm.at[srcb.at[tt]], growsb[s],
                                      gsem[s]).wait()
                for g in range(ne):
                    pltpu.make_async_copy(sd_hbm[2 * g].at[srcb.at[tt]],
                                          sdb[4 * g + 2 * s], gsem[s]).wait()
                    pltpu.make_async_copy(sd_hbm[2 * g + 1].at[dstb.at[tt]],
                                          sdb[4 * g + 2 * s + 1],
                                          gsem[s]).wait()

            def drain_scatter(s):
                pltpu.make_async_copy(growsb[s], acc.at[dstb.at[0]],
                                      ssem[s]).wait()

            fire(0, 0)

            @pl.loop(0, _CT // 2)
            def _(p):
                for s in (0, 1):
                    tt = p * 2 + s
                    drain(tt, s)
                    @pl.when(tt + 1 < _CT)
                    def _():
                        @pl.when(tt >= 1)
                        def _():
                            drain_scatter(1 - s)
                        fire(tt + 1, 1 - s)

                    for v in range(_SV):
                        sl = pl.ds(v * 16, 16)
                        valid = (ebase + c * cpe + tt * _TILE + v * 16
                                 + iota) < _E
                        dstv = dstb[tt, sl]
                        e1 = sdb[2 * s][sl] + sdb[2 * s + 1][sl]
                        e1 = jnp.where(e1 >= 0.0, e1, 0.2 * e1)
                        ex1 = jnp.where(valid, jnp.exp(e1), 0.0)
                        exb1[sl] = ex1
                        plsc.addupdate_scatter(denb[0], [dstv], ex1)
                        if two:
                            e3 = sdb[4 + 2 * s][sl] + sdb[4 + 2 * s + 1][sl]
                            e3 = jnp.where(e3 >= 0.0, e3, 0.2 * e3)
                            ex3 = jnp.where(valid, jnp.exp(e3), 0.0)
                            exb3[sl] = ex3
                            plsc.addupdate_scatter(denb[1], [dstv], ex3)

                    @pl.loop(0, _TILE)
                    def _(r, s=s):
                        rsp = jnp.full((16,), r, jnp.int32)
                        sp1 = plsc.load_gather(exb1, [rsp])
                        if two:
                            sp3 = plsc.load_gather(exb3, [rsp])
                            for cg in range(4):
                                growsb[s][r, pl.ds(cg * 16, 16)] = (
                                    growsb[s][r, pl.ds(cg * 16, 16)] * sp1)
                            for cg in range(4, 8):
                                growsb[s][r, pl.ds(cg * 16, 16)] = (
                                    growsb[s][r, pl.ds(cg * 16, 16)] * sp3)
                        else:
                            for cg in range(8):
                                growsb[s][r, pl.ds(cg * 16, 16)] = (
                                    growsb[s][r, pl.ds(cg * 16, 16)] * sp1)

                    pltpu.async_copy(growsb[s], acc.at[dstb.at[tt]],
                                     ssem[s], add=True)

            drain_scatter(0)
            drain_scatter(1)

        for g in range(ne):
            pltpu.sync_copy(denb[g],
                            den_hbm.at[pl.ds((wid * ne + g) * _N, _N)])

        plsc.subcore_barrier()

        @pl.loop(0, 3)
        def _(i):
            off = sid * 624 + i * 208
            pltpu.sync_copy(acc.at[pl.ds(off, 208)],
                            out_hbm.at[pl.ds(cid * _N + off, 208)])

        @pl.when(sid == 15)
        def _():
            pltpu.sync_copy(acc.at[pl.ds(9984, 16)],
                            out_hbm.at[pl.ds(cid * _N + 9984, 16)])

    return edge_kernel


_EDGE2 = _make_edge_kernel(True)
_EDGE1 = _make_edge_kernel(False)


def _read_body(mask_ref, hi_ref, lo_ref, out_ref, acc_ref):
    k = pl.program_id(1)
    nk = pl.num_programs(1)

    @pl.when(k == 0)
    def _():
        acc_ref[...] = jnp.zeros_like(acc_ref)

    mb = mask_ref[...]
    col = jax.lax.broadcasted_iota(jnp.int32, mb.shape, 1) + k * _BK
    mb = jnp.where(col < _N, mb, 0.0)
    mb16 = mb.astype(jnp.bfloat16)
    acc_ref[...] += (
        jnp.dot(mb16, hi_ref[...], preferred_element_type=jnp.float32)
        + jnp.dot(mb16, lo_ref[...], preferred_element_type=jnp.float32)
    )

    @pl.when(k == nk - 1)
    def _():
        out_ref[...] = acc_ref[...]


def _readout(mask, embcat):
    npad = 5 * _BK
    hi = embcat.astype(jnp.bfloat16)
    lo = (embcat - hi.astype(jnp.float32)).astype(jnp.bfloat16)
    hi_ext = jnp.zeros((npad, 256), jnp.bfloat16)
    hi_ext = hi_ext.at[:_N, :128].set(hi).at[:_N, 128].set(1.0)
    lo_ext = jnp.zeros((npad, 256), jnp.bfloat16).at[:_N, :128].set(lo)
    grid = (pl.cdiv(_N, _BM), npad // _BK)
    out = pl.pallas_call(
        _read_body,
        grid=grid,
        in_specs=[
            pl.BlockSpec((_BM, _BK), lambda i, k: (i, k)),
            pl.BlockSpec((_BK, 256), lambda i, k: (k, 0)),
            pl.BlockSpec((_BK, 256), lambda i, k: (k, 0)),
        ],
        out_specs=pl.BlockSpec((_BM, 256), lambda i, k: (i, 0)),
        out_shape=jax.ShapeDtypeStruct((_N, 256), jnp.float32),
        scratch_shapes=[pltpu.VMEM((_BM, 256), jnp.float32)],
    )(mask, hi_ext, lo_ext)
    return out[:, :128], out[:, 128]


def _bn(x, gamma, beta):
    return x / jnp.sqrt(1.0 + _BN_EPS) * gamma + beta


def kernel(feat, feat_a, adj, graph_neigh, W1, a_src1, a_dst1, W2, a_src2,
           a_dst2, bn1_gamma, bn1_beta, bn2_gamma, bn2_beta, Wb, bb):
    src, dst = adj[0], adj[1]
    pad = jnp.zeros((_EPAD - _E,), jnp.int32)
    nch = _NW * (_T // _CT)
    srcg = jnp.concatenate([src, pad]).reshape(nch, _CT, _TILE)
    dstg = jnp.concatenate([dst, pad]).reshape(nch, _CT, _TILE)

    h1 = feat @ W1
    h3 = feat_a @ W1
    hcat = jnp.concatenate([h1, h3], axis=1)
    out13, dens13 = _EDGE2(hcat, h1 @ a_src1, h1 @ a_dst1,
                           h3 @ a_src1, h3 @ a_dst1, srcg, dstg)
    agg = out13[:_N] + out13[_N:]
    d13 = dens13.reshape(_NW, 2, _N).sum(axis=0)
    den1 = d13[0] + 1e-16
    den3 = d13[1] + 1e-16
    z = _bn(agg[:, :64] / den1[:, None], bn1_gamma, bn1_beta)
    z_a = _bn(agg[:, 64:128] / den3[:, None], bn1_gamma, bn1_beta)
    hiden_emb = z

    h2 = z @ W2
    out2, dens2 = _EDGE1(h2, h2 @ a_src2, h2 @ a_dst2, srcg, dstg)
    agg2 = out2[:_N] + out2[_N:]
    den2 = dens2.reshape(_NW, _N).sum(axis=0) + 1e-16
    h_out = _bn(agg2 / den2[:, None], bn2_gamma, bn2_beta)

    emb = jax.nn.relu(z)
    emb_a = jax.nn.relu(z_a)
    embcat = jnp.concatenate([emb, emb_a], axis=1)

    vsum, rs = _readout(graph_neigh, embcat)
    gb = vsum / rs[:, None]
    n1 = jnp.maximum(jnp.linalg.norm(gb[:, :64], axis=1, keepdims=True), 1e-12)
    n2 = jnp.maximum(jnp.linalg.norm(gb[:, 64:], axis=1, keepdims=True), 1e-12)
    g = jax.nn.sigmoid(gb[:, :64] / n1)
    g_a = jax.nn.sigmoid(gb[:, 64:] / n2)

    t1 = emb @ Wb
    t2 = emb_a @ Wb
    ret = jnp.stack([jnp.sum(t1 * g, 1), jnp.sum(t2 * g, 1)], axis=1) + bb[0]
    ret_a = jnp.stack([jnp.sum(t2 * g_a, 1), jnp.sum(t1 * g_a, 1)], axis=1) + bb[0]
    return (hiden_emb, h_out, ret, ret_a)
